# baseline (speedup 1.0000x reference)
import functools

import jax
import jax.numpy as jnp
from jax import lax
from jax.experimental import pallas as pl
from jax.experimental.pallas import tpu as pltpu
from jax.experimental.pallas import tpu_sc as plsc

D_MODEL = 64
SCALE = 8.0
NC, NS, L = 2, 16, 16
NW = NC * NS
G = 128


def _sc_embed(idx, table, B):
    W = B // NW
    n_chunks = W // G

    mesh = plsc.VectorSubcoreMesh(core_axis_name="c", subcore_axis_name="s")

    @functools.partial(
        pl.kernel,
        out_type=jax.ShapeDtypeStruct((B, D_MODEL), jnp.float32),
        mesh=mesh,
        scratch_types=[
            pltpu.VMEM((G,), jnp.int32),
            pltpu.VMEM((G, D_MODEL), jnp.float32),
            pltpu.SemaphoreType.DMA,
        ],
        compiler_params=pltpu.CompilerParams(use_tc_tiling_on_sc=False),
    )
    def body(idx_hbm, table_hbm, out_hbm, idx_v, rows_v, sem):
        wid = lax.axis_index("s") * NC + lax.axis_index("c")
        base = wid * W

        def chunk(g, carry):
            off = base + g * G
            pltpu.sync_copy(idx_hbm.at[pl.ds(off, G)], idx_v)
            pltpu.async_copy(table_hbm.at[idx_v], rows_v, sem).wait()

            def row(r, c2):
                for c in range(D_MODEL // L):
                    rows_v[r, pl.ds(c * L, L)] = rows_v[r, pl.ds(c * L, L)] * SCALE
                return c2

            lax.fori_loop(0, G, row, 0)
            pltpu.sync_copy(rows_v, out_hbm.at[pl.ds(off, G)])
            return carry

        lax.fori_loop(0, n_chunks, chunk, 0)

    return body(idx, table)


def kernel(x, table):
    S, T = x.shape
    B = S * T
    idx = x.reshape(B).astype(jnp.int32)
    out = _sc_embed(idx, table, B)
    return out.reshape(S, T, D_MODEL)


# speedup vs baseline: 1.2636x; 1.2636x over previous
import functools

import jax
import jax.numpy as jnp
from jax import lax
from jax.experimental import pallas as pl
from jax.experimental.pallas import tpu as pltpu
from jax.experimental.pallas import tpu_sc as plsc

D_MODEL = 64
SCALE = 8.0
NC, NS, L = 2, 16, 16
NW = NC * NS
G = 128
C = 256
SUB = C // G
NBUF = 4


def _sc_embed(idx2d, table, B):
    W = B // NW
    n = W // C
    rows_per_chunk = SUB
    assert n % NBUF == 0

    mesh = plsc.VectorSubcoreMesh(core_axis_name="c", subcore_axis_name="s")

    @functools.partial(
        pl.kernel,
        out_type=jax.ShapeDtypeStruct((B, D_MODEL), jnp.float32),
        mesh=mesh,
        scratch_types=(
            [pltpu.VMEM((SUB, G), jnp.int32) for _ in range(NBUF)]
            + [pltpu.VMEM((C, D_MODEL), jnp.float32) for _ in range(NBUF)]
            + [pltpu.SemaphoreType.DMA] * (3 * NBUF)
        ),
        compiler_params=pltpu.CompilerParams(use_tc_tiling_on_sc=False),
    )
    def body(idx_hbm, table_hbm, out_hbm, *scratch):
        idxb = scratch[0:NBUF]
        rows = scratch[NBUF:2 * NBUF]
        sg = scratch[2 * NBUF:2 * NBUF + NBUF]
        so = scratch[2 * NBUF + NBUF:2 * NBUF + 2 * NBUF]
        si = scratch[2 * NBUF + 2 * NBUF:]

        wid = lax.axis_index("s") * NC + lax.axis_index("c")
        base = wid * W
        base_r = wid * (W // G)

        def fire_idx(g, b):
            gc = jnp.minimum(g, n - 1)
            pltpu.async_copy(
                idx_hbm.at[pl.ds(base_r + gc * rows_per_chunk, SUB)],
                idxb[b], si[b])

        def wait_idx(b):
            pltpu.make_async_copy(
                idx_hbm.at[pl.ds(0, SUB)], idxb[b], si[b]).wait()

        def fire_gather(b):
            for j in range(SUB):
                pltpu.async_copy(
                    table_hbm.at[idxb[b].at[j]],
                    rows[b].at[pl.ds(j * G, G)], sg[b])

        def wait_rows_sem(sem, b):
            pltpu.make_async_copy(
                table_hbm.at[pl.ds(0, C)], rows[b], sem[b]).wait()

        fire_idx(0, 0)
        fire_idx(1, 1)
        for r in range(1, NBUF):
            pltpu.async_copy(table_hbm.at[pl.ds(0, C)], rows[r], so[r])
        wait_idx(0)
        fire_gather(0)

        def step(g, b, bn, bi):
            wait_idx(bn)
            wait_rows_sem(so, bn)
            fire_gather(bn)
            wait_rows_sem(sg, b)
            fire_idx(g + 2, bi)

            @plsc.parallel_loop(0, C, 1, unroll=4)
            def _scale(r):
                for c in range(D_MODEL // L):
                    rows[b][r, pl.ds(c * L, L)] = (
                        rows[b][r, pl.ds(c * L, L)] * SCALE)

            pltpu.async_copy(rows[b], out_hbm.at[pl.ds(base + g * C, C)],
                             so[b])

        def quad(p, carry):
            for bb in range(NBUF):
                step(p * NBUF + bb, bb, (bb + 1) % NBUF, (bb + 2) % NBUF)
            return carry

        lax.fori_loop(0, n // NBUF, quad, 0)

        for r in range(1, NBUF):
            wait_rows_sem(so, r)
        wait_rows_sem(sg, 0)
        wait_idx(1)

    return body(idx2d, table)


def kernel(x, table):
    S, T = x.shape
    B = S * T
    idx2d = x.reshape(B // G, G).astype(jnp.int32)
    out = _sc_embed(idx2d, table, B)
    return out.reshape(S, T, D_MODEL)
